# baseline (speedup 1.0000x reference)
import functools

import jax
import jax.numpy as jnp
from jax import lax
from jax.experimental import pallas as pl
from jax.experimental.pallas import tpu as pltpu
from jax.experimental.pallas import tpu_sc as plsc

B = 1024
J = 26
C = 1000
ROW = J * C
NW = 32
ROWS_PER_W = B // NW
R = 4
NCHUNK = ROWS_PER_W // R

_mesh = plsc.VectorSubcoreMesh(core_axis_name="c", subcore_axis_name="s")


@functools.partial(
    pl.kernel,
    mesh=_mesh,
    out_type=jax.ShapeDtypeStruct((B * ROW,), jnp.int32),
    scratch_types=[
        pltpu.VMEM((ROWS_PER_W * J,), jnp.int32),
        pltpu.VMEM((R * ROW,), jnp.int32),
    ],
    compiler_params=pltpu.CompilerParams(needs_layout_passes=False),
)
def _onehot_sc(x_hbm, zeros_hbm, out_hbm, xv, buf):
    wid = lax.axis_index("s") * 2 + lax.axis_index("c")
    base_row = wid * ROWS_PER_W
    pltpu.sync_copy(x_hbm.at[pl.ds(base_row * J, ROWS_PER_W * J)], xv)
    pltpu.sync_copy(zeros_hbm, buf)

    offs = lax.broadcasted_iota(jnp.int32, (16,), 0) * C
    ones = jnp.full((16,), 1, jnp.int32)
    zeros_v = jnp.zeros((16,), jnp.int32)

    for chunk in range(NCHUNK):
        idx_vecs = []
        for r in range(R):
            row = chunk * R + r
            xa = xv[pl.ds(row * J, 16)]
            xb = xv[pl.ds(row * J + (J - 16), 16)]
            ia = xa + offs + r * ROW
            ib = xb + offs + (r * ROW + (J - 16) * C)
            plsc.store_scatter(buf, [ia], ones)
            plsc.store_scatter(buf, [ib], ones)
            idx_vecs.append((ia, ib))
        dst = out_hbm.at[pl.ds((base_row + chunk * R) * ROW, R * ROW)]
        pltpu.sync_copy(buf, dst)
        for ia, ib in idx_vecs:
            plsc.store_scatter(buf, [ia], zeros_v)
            plsc.store_scatter(buf, [ib], zeros_v)


def kernel(x):
    xf = x.reshape(-1).astype(jnp.int32)
    zeros = jnp.zeros((R * ROW,), jnp.int32)
    out = _onehot_sc(xf, zeros)
    return out.reshape(B, ROW)
